# Initial kernel scaffold; baseline (speedup 1.0000x reference)
import jax
import jax.numpy as jnp
from jax.experimental import pallas as pl


def kernel(trg_tokens_logprobas, target_token_idxs):
    raise NotImplementedError("write your pallas kernel here")


# speedup vs baseline: 6.6054x; 6.6054x over previous
import functools
import math

import jax
import jax.numpy as jnp
from jax.experimental import pallas as pl
from jax.experimental.pallas import tpu as pltpu

V = 32000
SMOOTH = 0.1
CONF = 1.0 - SMOOTH
SV = SMOOTH / (V - 1)
K_ROW = CONF * math.log(CONF) + (V - 2) * SV * math.log(SV)

BR = 256
BV = 6400


def _loss_body(t_ref, x_ref, o_ref):
    ri = pl.program_id(0)
    vi = pl.program_id(1)

    @pl.when((ri == 0) & (vi == 0))
    def _():
        o_ref[0, 0] = 0.0

    x = x_ref[...]
    t2 = t_ref[0]
    col = jax.lax.broadcasted_iota(jnp.int32, (BR, BV), 1) + vi * BV
    w = jnp.where(col == t2, -CONF, -SV)
    w = jnp.where(col == 0, 0.0, w)
    w = jnp.where(t2 == 0, 0.0, w)
    partial = jnp.sum(w * x)

    @pl.when(vi == 0)
    def _():
        nonpad = jnp.sum((t2 != 0).astype(jnp.float32))
        o_ref[0, 0] += K_ROW * nonpad

    o_ref[0, 0] += partial


@functools.partial(jax.jit, static_argnames=())
def _loss(x2d, t3d):
    nr = x2d.shape[0] // BR
    nv = V // BV
    out = pl.pallas_call(
        _loss_body,
        grid=(nr, nv),
        in_specs=[
            pl.BlockSpec((1, BR, 1), lambda ri, vi: (ri, 0, 0)),
            pl.BlockSpec((BR, BV), lambda ri, vi: (ri, vi)),
        ],
        out_specs=pl.BlockSpec(
            (1, 1), lambda ri, vi: (0, 0), memory_space=pltpu.SMEM
        ),
        out_shape=jax.ShapeDtypeStruct((1, 1), jnp.float32),
    )(t3d, x2d)
    return out[0, 0]


def kernel(trg_tokens_logprobas, target_token_idxs):
    B, S, Vv = trg_tokens_logprobas.shape
    x2d = trg_tokens_logprobas.reshape(B * S, Vv)
    t = target_token_idxs.astype(jnp.int32).reshape(B * S)
    nr = (B * S) // BR
    t3d = t.reshape(nr, BR, 1)
    return _loss(x2d, t3d)
